# baseline (speedup 1.0000x reference)
import functools

import jax
import jax.numpy as jnp
from jax import lax
from jax.experimental import pallas as pl
from jax.experimental.pallas import tpu as pltpu
from jax.experimental.pallas import tpu_sc as plsc

B = 2
N = 10000
E = 160000
D = 128

NC = 2
NS = 16
NW = NC * NS
ROWS_W = (B * E) // NW
CH = 80
NCH = ROWS_W // CH


def _build_gather():
  mesh = plsc.VectorSubcoreMesh(core_axis_name="c", subcore_axis_name="s")

  @functools.partial(
      pl.kernel,
      mesh=mesh,
      out_type=jax.ShapeDtypeStruct((B * E, D), jnp.float32),
      scratch_types=[
          pltpu.VMEM((ROWS_W,), jnp.int32),
          pltpu.VMEM((4, CH, D), jnp.float32),
          pltpu.VMEM_SHARED((N, D), jnp.float32),
          pltpu.SemaphoreType.DMA,
          pltpu.SemaphoreType.DMA,
          pltpu.SemaphoreType.DMA,
          pltpu.SemaphoreType.DMA,
          pltpu.SemaphoreType.DMA,
          pltpu.SemaphoreType.DMA,
          pltpu.SemaphoreType.DMA,
          pltpu.SemaphoreType.DMA,
          pltpu.SemaphoreType.DMA,
      ],
  )
  def gather_kernel(hv, idx_hbm, out_hbm, idx_v, bufs, table_sp, gs0, gs1,
                    gs2, gs3, ss0, ss1, ss2, ss3, tsem):
    c = lax.axis_index("c")
    s = lax.axis_index("s")
    base = c * E + s * ROWS_W
    rows_stage = (N // NS) // 8 * 8
    tail = N - NS * rows_stage
    stage_main = pltpu.make_async_copy(
        hv.at[c].at[pl.ds(s * rows_stage, rows_stage)],
        table_sp.at[pl.ds(s * rows_stage, rows_stage)],
        tsem,
    )
    stage_main.start()

    @pl.when(s == 0)
    def _():
      pltpu.make_async_copy(
          hv.at[c].at[pl.ds(NS * rows_stage, tail)],
          table_sp.at[pl.ds(NS * rows_stage, tail)],
          tsem,
      ).start()

    pltpu.sync_copy(idx_hbm.at[pl.ds(base, ROWS_W)], idx_v)

    def stage_wait():
      stage_main.wait()

      @pl.when(s == 0)
      def _():
        pltpu.make_async_copy(
            hv.at[c].at[pl.ds(NS * rows_stage, tail)],
            table_sp.at[pl.ds(NS * rows_stage, tail)],
            tsem,
        ).wait()

      plsc.subcore_barrier()

    gsems = (gs0, gs1, gs2, gs3)
    ssems = (ss0, ss1, ss2, ss3)

    def run(table, table_hbm):
      def g_start(j, b, tab=None):
        pltpu.make_async_copy(
            (table if tab is None else tab).at[idx_v.at[pl.ds(j * CH, CH)]],
            bufs.at[b],
            gsems[b],
        ).start()

      def g_wait(b):
        pltpu.make_async_copy(
            table.at[idx_v.at[pl.ds(0, CH)]], bufs.at[b], gsems[b]
        ).wait()

      def s_start(j, b):
        pltpu.make_async_copy(
            bufs.at[b], out_hbm.at[pl.ds(base + j * CH, CH)], ssems[b]
        ).start()

      def s_wait(b):
        pltpu.make_async_copy(
            bufs.at[b], out_hbm.at[pl.ds(base, CH)], ssems[b]
        ).wait()

      g_start(0, 0, table_hbm)
      g_start(1, 1, table_hbm)
      g_start(2, 2, table_hbm)
      stage_wait()
      g_wait(0)
      s_start(0, 0)
      g_start(3, 3)
      g_wait(1)
      s_start(1, 1)

      def body(jj, carry):
        m0 = 2 + 4 * jj
        for k in range(4):
          m = m0 + k
          b = (2 + k) % 4
          bn = (b + 2) % 4
          g_wait(b)
          s_start(m, b)
          s_wait(bn)
          g_start(m + 2, bn)
        return carry

      lax.fori_loop(0, (NCH - 5) // 4, body, 0)

      g_wait(2)
      s_start(NCH - 3, 2)
      s_wait(0)
      g_start(NCH - 1, 0)
      g_wait(3)
      s_start(NCH - 2, 3)
      g_wait(0)
      s_start(NCH - 1, 0)
      s_wait(1)
      s_wait(2)
      s_wait(3)
      s_wait(0)

    run(table_sp, hv.at[c])

  return gather_kernel


_gather = _build_gather()


@jax.jit
def kernel(h_v, edge_input):
  idx = edge_input[:, :, 1].astype(jnp.int32).reshape(-1)
  out = _gather(h_v, idx)
  return out.reshape(B, E, D)


# speedup vs baseline: 1.0199x; 1.0199x over previous
import functools

import jax
import jax.numpy as jnp
from jax import lax
from jax.experimental import pallas as pl
from jax.experimental.pallas import tpu as pltpu
from jax.experimental.pallas import tpu_sc as plsc

B = 2
N = 10000
E = 160000
D = 128

NC = 2
NS = 16
NW = NC * NS
ROWS_W = (B * E) // NW
CH = 80
NCH = ROWS_W // CH


def _build_gather():
  mesh = plsc.VectorSubcoreMesh(core_axis_name="c", subcore_axis_name="s")

  @functools.partial(
      pl.kernel,
      mesh=mesh,
      out_type=jax.ShapeDtypeStruct((B * E, D), jnp.float32),
      scratch_types=[
          pltpu.VMEM((ROWS_W,), jnp.int32),
          pltpu.VMEM((4, CH, D), jnp.float32),
          pltpu.VMEM_SHARED((N, D), jnp.float32),
          pltpu.SemaphoreType.DMA,
          pltpu.SemaphoreType.DMA,
          pltpu.SemaphoreType.DMA,
          pltpu.SemaphoreType.DMA,
          pltpu.SemaphoreType.DMA,
          pltpu.SemaphoreType.DMA,
          pltpu.SemaphoreType.DMA,
          pltpu.SemaphoreType.DMA,
          pltpu.SemaphoreType.DMA,
      ],
  )
  def gather_kernel(hv, idx_hbm, out_hbm, idx_v, bufs, table_sp, gs0, gs1,
                    gs2, gs3, ss0, ss1, ss2, ss3, tsem):
    c = lax.axis_index("c")
    s = lax.axis_index("s")
    base = c * E + s * ROWS_W
    rows_stage = (N // NS) // 8 * 8
    tail = N - NS * rows_stage
    stage_main = pltpu.make_async_copy(
        hv.at[c].at[pl.ds(s * rows_stage, rows_stage)],
        table_sp.at[pl.ds(s * rows_stage, rows_stage)],
        tsem,
    )
    stage_main.start()

    @pl.when(s == 0)
    def _():
      pltpu.make_async_copy(
          hv.at[c].at[pl.ds(NS * rows_stage, tail)],
          table_sp.at[pl.ds(NS * rows_stage, tail)],
          tsem,
      ).start()

    pltpu.sync_copy(idx_hbm.at[pl.ds(base, ROWS_W)], idx_v)

    def stage_wait():
      stage_main.wait()

      @pl.when(s == 0)
      def _():
        pltpu.make_async_copy(
            hv.at[c].at[pl.ds(NS * rows_stage, tail)],
            table_sp.at[pl.ds(NS * rows_stage, tail)],
            tsem,
        ).wait()

      plsc.subcore_barrier()

    gsems = (gs0, gs1, gs2, gs3)
    ssems = (ss0, ss1, ss2, ss3)

    def run(table, table_hbm):
      def g_start(j, b, tab=None):
        pltpu.make_async_copy(
            (table if tab is None else tab).at[idx_v.at[pl.ds(j * CH, CH)]],
            bufs.at[b],
            gsems[b],
        ).start()

      def g_wait(b):
        pltpu.make_async_copy(
            table.at[idx_v.at[pl.ds(0, CH)]], bufs.at[b], gsems[b]
        ).wait()

      def s_start(j, b):
        pltpu.make_async_copy(
            bufs.at[b], out_hbm.at[pl.ds(base + j * CH, CH)], ssems[b]
        ).start()

      def s_wait(b):
        pltpu.make_async_copy(
            bufs.at[b], out_hbm.at[pl.ds(base, CH)], ssems[b]
        ).wait()

      g_start(0, 0, table_hbm)
      g_start(1, 1, table_hbm)
      g_start(2, 2, table_hbm)
      g_wait(0)
      s_start(0, 0)
      g_start(3, 3, table_hbm)
      g_wait(1)
      s_start(1, 1)
      stage_wait()

      def body(jj, carry):
        m0 = 2 + 4 * jj
        for k in range(4):
          m = m0 + k
          b = (2 + k) % 4
          bn = (b + 2) % 4
          s_wait(bn)
          g_start(m + 2, bn)
          g_wait(b)
          s_start(m, b)
        return carry

      lax.fori_loop(0, (NCH - 5) // 4, body, 0)

      s_wait(0)
      g_start(NCH - 1, 0)
      g_wait(2)
      s_start(NCH - 3, 2)
      g_wait(3)
      s_start(NCH - 2, 3)
      g_wait(0)
      s_start(NCH - 1, 0)
      s_wait(1)
      s_wait(2)
      s_wait(3)
      s_wait(0)

    run(table_sp, hv.at[c])

  return gather_kernel


_gather = _build_gather()


@jax.jit
def kernel(h_v, edge_input):
  idx = edge_input[:, :, 1].astype(jnp.int32).reshape(-1)
  out = _gather(h_v, idx)
  return out.reshape(B, E, D)
